# Initial kernel scaffold; baseline (speedup 1.0000x reference)
import jax
import jax.numpy as jnp
from jax.experimental import pallas as pl


def kernel(x_t, t, condition, edge_index, edge_attr, batch, params):
    raise NotImplementedError("write your pallas kernel here")


# speedup vs baseline: 1.9830x; 1.9830x over previous
import functools
import math

import jax
import jax.numpy as jnp
from jax import lax
from jax.experimental import pallas as pl
from jax.experimental.pallas import tpu as pltpu
from jax.experimental.pallas import tpu_sc as plsc

H = 64
_NC, _NS, _L = 2, 16, 16


def _silu(x):
    return x / (1.0 + jnp.exp(-x))


@functools.partial(jax.jit, static_argnames=("N", "E"))
def _sc_scatter(h, ef, src, dst, *, N, E):
    HALF = N // _NC
    ROWS = 26112
    TROWS = ROWS // _NS
    ZCH = 136
    ZIT = TROWS // ZCH
    WR = 1568
    WR_LAST = HALF - (_NS - 1) * WR
    K = 80
    EPT = E // _NS
    ITERS = EPT // K

    mesh = plsc.VectorSubcoreMesh(core_axis_name="c", subcore_axis_name="s",
                                  num_cores=_NC, num_subcores=_NS)

    @functools.partial(
        pl.kernel,
        mesh=mesh,
        out_type=jax.ShapeDtypeStruct((N, H), jnp.float32),
        compiler_params=pltpu.CompilerParams(use_tc_tiling_on_sc=False),
        scratch_types=[
            pltpu.VMEM((ZCH, H), jnp.float32),
            pltpu.VMEM((K,), jnp.int32),
            pltpu.VMEM((K,), jnp.int32),
            pltpu.VMEM((K,), jnp.int32),
            pltpu.VMEM((K, H), jnp.float32),
            pltpu.VMEM((K, H), jnp.float32),
            pltpu.VMEM_SHARED((ROWS, H), jnp.float32),
            pltpu.SemaphoreType.DMA,
        ],
    )
    def scat(h_hbm, ef_hbm, src_hbm, dst_hbm, out_hbm, zbuf, srcv, dstv, idxv,
             hrv, efv, aggs, sem):
        c = lax.axis_index("c")
        s = lax.axis_index("s")

        zero16 = jnp.zeros((_L,), jnp.float32)

        def zfill(i, carry):
            r = i // (H // _L)
            q = i % (H // _L)
            zbuf[r, pl.ds(q * _L, _L)] = zero16
            return carry

        lax.fori_loop(0, ZCH * (H // _L), zfill, 0)

        def zcopy(i, carry):
            pltpu.sync_copy(zbuf, aggs.at[pl.ds(s * TROWS + i * ZCH, ZCH)])
            return carry

        lax.fori_loop(0, ZIT, zcopy, 0)
        plsc.subcore_barrier()

        base_row = c * HALF

        def step(i, carry):
            eb = s * EPT + i * K
            pltpu.sync_copy(src_hbm.at[pl.ds(eb, K)], srcv)
            cp = pltpu.async_copy(h_hbm.at[srcv], hrv, sem)
            pltpu.sync_copy(dst_hbm.at[pl.ds(eb, K)], dstv)
            pltpu.sync_copy(ef_hbm.at[pl.ds(eb, K)], efv)

            def ixb(j, cr):
                d = dstv[pl.ds(j * _L, _L)]
                loc = d - base_row
                ok = (loc >= 0) & (loc < HALF)
                idxv[pl.ds(j * _L, _L)] = jnp.where(ok, loc, HALF)
                return cr

            lax.fori_loop(0, K // _L, ixb, 0)
            cp.wait()

            def row(r, cr):
                for q in range(H // _L):
                    sl = pl.ds(q * _L, _L)
                    efv[r, sl] = jnp.maximum(hrv[r, sl] + efv[r, sl], 0.0)
                return cr

            lax.fori_loop(0, K, row, 0)
            pltpu.sync_copy(efv, aggs.at[idxv], add=True)
            return carry

        lax.fori_loop(0, ITERS, step, 0)
        plsc.subcore_barrier()

        @pl.when(s < _NS - 1)
        def _():
            pltpu.sync_copy(
                aggs.at[pl.ds(s * WR, WR)],
                out_hbm.at[pl.ds(c * HALF + s * WR, WR)],
            )

        @pl.when(s == _NS - 1)
        def _():
            pltpu.sync_copy(
                aggs.at[pl.ds((_NS - 1) * WR, WR_LAST)],
                out_hbm.at[pl.ds(c * HALF + (_NS - 1) * WR, WR_LAST)],
            )

    return scat(h, ef, src, dst)


_NBLK = 1000
_EBLK = 8000


def _dot(a, b):
    return jnp.dot(a, b, preferred_element_type=jnp.float32)


def _node_encoder(x_in, batch2, te, w1, b1, w2, b2, tw, tb):
    N = x_in.shape[0]
    grid = N // _NBLK

    def body(xb, bb, teb, w1b, b1b, w2b, b2b, twb, tbb, out):
        pre = _dot(xb[...], w1b[...]) + b1b[...]
        hcur = _dot(_silu(pre), w2b[...]) + b2b[...]
        tf = _dot(_silu(teb[...]), twb[...]) + tbb[...]
        bbv = bb[...]
        for k in range(16):
            hcur += jnp.where(bbv == k, 1.0, 0.0) * tf[k:k + 1, :]
        out[...] = hcur

    full = lambda shape: pl.BlockSpec(shape, lambda i: (0, 0))
    return pl.pallas_call(
        body,
        grid=(grid,),
        in_specs=[
            pl.BlockSpec((_NBLK, 7), lambda i: (i, 0)),
            pl.BlockSpec((_NBLK, 1), lambda i: (i, 0)),
            full((16, 32)),
            full((7, H)), full((1, H)),
            full((H, H)), full((1, H)),
            full((32, H)), full((1, H)),
        ],
        out_specs=pl.BlockSpec((_NBLK, H), lambda i: (i, 0)),
        out_shape=jax.ShapeDtypeStruct((N, H), jnp.float32),
    )(x_in, batch2, te, w1, b1, w2, b2, tw, tb)


def _edge_mlp(ea, w1, b1, w2, b2):
    E = ea.shape[0]
    grid = E // _EBLK

    def body(eb, w1b, b1b, w2b, b2b, out):
        hid = _silu(_dot(eb[...], w1b[...]) + b1b[...])
        out[...] = _dot(hid, w2b[...]) + b2b[...]

    full = lambda shape: pl.BlockSpec(shape, lambda i: (0, 0))
    return pl.pallas_call(
        body,
        grid=(grid,),
        in_specs=[
            pl.BlockSpec((_EBLK, 2), lambda i: (i, 0)),
            full((2, H)), full((1, H)), full((H, H)), full((1, H)),
        ],
        out_specs=pl.BlockSpec((_EBLK, H), lambda i: (i, 0)),
        out_shape=jax.ShapeDtypeStruct((E, H), jnp.float32),
    )(ea, w1, b1, w2, b2)


def _layer_mlp(h, aggp, w1, b1, w2, b2):
    N = h.shape[0]
    grid = N // _NBLK

    def body(hb, ab, w1b, b1b, w2b, b2b, out):
        z = hb[...] + ab[...]
        hid = _silu(_dot(z, w1b[...]) + b1b[...])
        out[...] = _silu(_dot(hid, w2b[...]) + b2b[...])

    full = lambda shape: pl.BlockSpec(shape, lambda i: (0, 0))
    return pl.pallas_call(
        body,
        grid=(grid,),
        in_specs=[
            pl.BlockSpec((_NBLK, H), lambda i: (i, 0)),
            pl.BlockSpec((_NBLK, H), lambda i: (i, 0)),
            full((H, H)), full((1, H)), full((H, H)), full((1, H)),
        ],
        out_specs=pl.BlockSpec((_NBLK, H), lambda i: (i, 0)),
        out_shape=jax.ShapeDtypeStruct((N, H), jnp.float32),
    )(h, aggp, w1, b1, w2, b2)


def _final_mlp(h, aggp, w1, b1, w2, b2, fw1, fb1, fw2, fb2):
    N = h.shape[0]
    grid = N // _NBLK

    def body(hb, ab, w1b, b1b, w2b, b2b, fw1b, fb1b, fw2b, fb2b, out):
        z = hb[...] + ab[...]
        hid = _silu(_dot(z, w1b[...]) + b1b[...])
        h3 = _silu(_dot(hid, w2b[...]) + b2b[...])
        fh = _silu(_dot(h3, fw1b[...]) + fb1b[...])
        out[...] = _dot(fh, fw2b[...]) + fb2b[...]

    full = lambda shape: pl.BlockSpec(shape, lambda i: (0, 0))
    return pl.pallas_call(
        body,
        grid=(grid,),
        in_specs=[
            pl.BlockSpec((_NBLK, H), lambda i: (i, 0)),
            pl.BlockSpec((_NBLK, H), lambda i: (i, 0)),
            full((H, H)), full((1, H)), full((H, H)), full((1, H)),
            full((H, H)), full((1, H)), full((H, 1)), full((1, 1)),
        ],
        out_specs=pl.BlockSpec((_NBLK, 1), lambda i: (i, 0)),
        out_shape=jax.ShapeDtypeStruct((N, 1), jnp.float32),
    )(h, aggp, w1, b1, w2, b2, fw1, fb1, fw2, fb2)


def kernel(x_t, t, condition, edge_index, edge_attr, batch, params):
    N = x_t.shape[0]
    E = edge_index.shape[1]
    p = params

    half = 16
    freq = jnp.exp(jnp.arange(half, dtype=jnp.float32)
                   * (-math.log(10000.0) / (half - 1)))
    ang = t[:, None].astype(jnp.float32) * freq[None, :]
    te = jnp.concatenate([jnp.sin(ang), jnp.cos(ang)], axis=-1)

    r1 = lambda v: v.reshape(1, -1)
    batch2 = batch.reshape(N, 1)

    x_in = jnp.concatenate([x_t, condition], axis=-1)
    h = _node_encoder(
        x_in, batch2, te,
        p['node_w1'], r1(p['node_b1']),
        p['node_w2'], r1(p['node_b2']),
        p['time_w'], r1(p['time_b']),
    )
    ef = _edge_mlp(edge_attr, p['edge_w1'], r1(p['edge_b1']),
                   p['edge_w2'], r1(p['edge_b2']))

    src, dst = edge_index[0], edge_index[1]
    for name in ('c1', 'c2'):
        aggp = _sc_scatter(h, ef, src, dst, N=N, E=E)
        h = _layer_mlp(h, aggp, p[name + '_w1'], r1(p[name + '_b1']),
                       p[name + '_w2'], r1(p[name + '_b2']))

    aggp = _sc_scatter(h, ef, src, dst, N=N, E=E)
    return _final_mlp(h, aggp, p['c3_w1'], r1(p['c3_b1']),
                      p['c3_w2'], r1(p['c3_b2']),
                      p['f_w1'], r1(p['f_b1']),
                      p['f_w2'], jnp.full((1, 1), p['f_b2'][0]))


# speedup vs baseline: 3.6619x; 1.8467x over previous
import functools
import math

import jax
import jax.numpy as jnp
from jax import lax
from jax.experimental import pallas as pl
from jax.experimental.pallas import tpu as pltpu
from jax.experimental.pallas import tpu_sc as plsc

H = 64
_NC, _NS, _L = 2, 16, 16


def _silu(x):
    return x / (1.0 + jnp.exp(-x))


@functools.partial(jax.jit, static_argnames=("N", "E"))
def _sc_scatter(h, ef, src, dst, *, N, E):
    HALF = N // _NC
    ROWS = 25088
    TROWS = ROWS // _NS
    ZCH = 56
    ZIT = TROWS // ZCH
    WR = 1568
    WR_LAST = HALF - (_NS - 1) * WR
    K = 80
    EPT = E // _NS
    NCH = EPT // K
    NCHP = NCH - 1
    NP = NCHP // 2

    mesh = plsc.VectorSubcoreMesh(core_axis_name="c", subcore_axis_name="s",
                                  num_cores=_NC, num_subcores=_NS)

    @functools.partial(
        pl.kernel,
        mesh=mesh,
        out_type=jax.ShapeDtypeStruct((N, H), jnp.float32),
        compiler_params=pltpu.CompilerParams(use_tc_tiling_on_sc=False),
        scratch_types=[
            pltpu.VMEM((ZCH, H), jnp.float32),
            [pltpu.VMEM((K,), jnp.int32) for _ in range(2)],
            [pltpu.VMEM((K,), jnp.int32) for _ in range(2)],
            [pltpu.VMEM((K,), jnp.int32) for _ in range(2)],
            [pltpu.VMEM((K, H), jnp.float32) for _ in range(2)],
            [pltpu.VMEM((K, H), jnp.float32) for _ in range(2)],
            pltpu.VMEM_SHARED((ROWS, H), jnp.float32),
            [pltpu.SemaphoreType.DMA for _ in range(2)],
            [pltpu.SemaphoreType.DMA for _ in range(2)],
            [pltpu.SemaphoreType.DMA for _ in range(2)],
            [pltpu.SemaphoreType.DMA for _ in range(2)],
        ],
    )
    def scat(h_hbm, ef_hbm, src_hbm, dst_hbm, out_hbm, zbuf, srcs, dsts, idxs,
             hrs, efs, aggs, semi, semg, seme, sems):
        c = lax.axis_index("c")
        s = lax.axis_index("s")

        zero16 = jnp.zeros((_L,), jnp.float32)

        def zfill(i, carry):
            r = i // (H // _L)
            q = i % (H // _L)
            zbuf[r, pl.ds(q * _L, _L)] = zero16
            return carry

        lax.fori_loop(0, ZCH * (H // _L), zfill, 0)

        def zcopy(i, carry):
            pltpu.sync_copy(zbuf, aggs.at[pl.ds(s * TROWS + i * ZCH, ZCH)])
            return carry

        lax.fori_loop(0, ZIT, zcopy, 0)
        plsc.subcore_barrier()

        base_row = c * HALF
        ebase = s * EPT

        def issue_idx(ch, b):
            eb = ebase + ch * K
            pltpu.async_copy(src_hbm.at[pl.ds(eb, K)], srcs[b], semi[b])
            pltpu.async_copy(dst_hbm.at[pl.ds(eb, K)], dsts[b], semi[b])

        def drain_idx(b):
            pltpu.make_async_copy(src_hbm.at[pl.ds(0, K)], srcs[b],
                                  semi[b]).wait()
            pltpu.make_async_copy(dst_hbm.at[pl.ds(0, K)], dsts[b],
                                  semi[b]).wait()

        def compute_idx(dref, iref, n):
            def ixb(j, cr):
                d = dref[pl.ds(j * _L, _L)]
                loc = d - base_row
                ok = (loc >= 0) & (loc < HALF)
                iref[pl.ds(j * _L, _L)] = jnp.where(ok, loc, HALF)
                return cr
            lax.fori_loop(0, n // _L, ixb, 0)

        def compute_msg(href, eref, mref, n):
            def row(r, cr):
                for q in range(H // _L):
                    sl = pl.ds(q * _L, _L)
                    mref[r, sl] = jnp.maximum(href[r, sl] + eref[r, sl], 0.0)
                return cr
            lax.fori_loop(0, n, row, 0)

        issue_idx(0, 0)
        drain_idx(0)
        pltpu.async_copy(h_hbm.at[srcs[0]], hrs[0], semg[0])
        pltpu.async_copy(ef_hbm.at[pl.ds(ebase, K)], efs[0], seme[0])
        issue_idx(1, 1)

        def pair(j, carry):
            for b in (0, 1):
                ch = 2 * j + b
                nb = 1 - b

                compute_idx(dsts[b], idxs[b], K)

                @pl.when(ch + 1 < NCHP)
                def _():
                    @pl.when(ch >= 1)
                    def _():
                        pltpu.make_async_copy(efs[nb], aggs.at[pl.ds(0, K)],
                                              sems[nb]).wait()
                    drain_idx(nb)
                    pltpu.async_copy(h_hbm.at[srcs[nb]], hrs[nb], semg[nb])
                    pltpu.async_copy(
                        ef_hbm.at[pl.ds(ebase + (ch + 1) * K, K)],
                        efs[nb], seme[nb])

                pltpu.make_async_copy(h_hbm.at[pl.ds(0, K)], hrs[b],
                                      semg[b]).wait()
                pltpu.make_async_copy(ef_hbm.at[pl.ds(0, K)], efs[b],
                                      seme[b]).wait()

                @pl.when(ch + 2 < NCHP)
                def _():
                    issue_idx(ch + 2, b)

                compute_msg(hrs[b], efs[b], efs[b], K)
                pltpu.async_copy(efs[b], aggs.at[idxs[b]], sems[b], add=True)
            return carry

        lax.fori_loop(0, NP, pair, 0)
        pltpu.make_async_copy(efs[0], aggs.at[pl.ds(0, K)], sems[0]).wait()
        pltpu.make_async_copy(efs[1], aggs.at[pl.ds(0, K)], sems[1]).wait()

        tb = ebase + NCHP * K
        pltpu.sync_copy(src_hbm.at[pl.ds(tb, K)], srcs[0])
        cp = pltpu.async_copy(h_hbm.at[srcs[0]], hrs[0], semg[0])
        pltpu.sync_copy(dst_hbm.at[pl.ds(tb, K)], dsts[0])
        pltpu.sync_copy(ef_hbm.at[pl.ds(tb, K)], efs[0])
        compute_idx(dsts[0], idxs[0], K)
        cp.wait()
        compute_msg(hrs[0], efs[0], efs[0], K)
        pltpu.sync_copy(efs[0], aggs.at[idxs[0]], add=True)
        plsc.subcore_barrier()

        @pl.when(s < _NS - 1)
        def _():
            pltpu.sync_copy(
                aggs.at[pl.ds(s * WR, WR)],
                out_hbm.at[pl.ds(c * HALF + s * WR, WR)],
            )

        @pl.when(s == _NS - 1)
        def _():
            pltpu.sync_copy(
                aggs.at[pl.ds((_NS - 1) * WR, WR_LAST)],
                out_hbm.at[pl.ds(c * HALF + (_NS - 1) * WR, WR_LAST)],
            )

    return scat(h, ef, src, dst)


_NBLK = 1000
_EBLK = 8000


def _dot(a, b):
    return jnp.dot(a, b, preferred_element_type=jnp.float32)


def _node_encoder(x_in, batch2, te, w1, b1, w2, b2, tw, tb):
    N = x_in.shape[0]
    grid = N // _NBLK

    def body(xb, bb, teb, w1b, b1b, w2b, b2b, twb, tbb, out):
        pre = _dot(xb[...], w1b[...]) + b1b[...]
        hcur = _dot(_silu(pre), w2b[...]) + b2b[...]
        tf = _dot(_silu(teb[...]), twb[...]) + tbb[...]
        bbv = bb[...]
        for k in range(16):
            hcur += jnp.where(bbv == k, 1.0, 0.0) * tf[k:k + 1, :]
        out[...] = hcur

    full = lambda shape: pl.BlockSpec(shape, lambda i: (0, 0))
    return pl.pallas_call(
        body,
        grid=(grid,),
        in_specs=[
            pl.BlockSpec((_NBLK, 7), lambda i: (i, 0)),
            pl.BlockSpec((_NBLK, 1), lambda i: (i, 0)),
            full((16, 32)),
            full((7, H)), full((1, H)),
            full((H, H)), full((1, H)),
            full((32, H)), full((1, H)),
        ],
        out_specs=pl.BlockSpec((_NBLK, H), lambda i: (i, 0)),
        out_shape=jax.ShapeDtypeStruct((N, H), jnp.float32),
    )(x_in, batch2, te, w1, b1, w2, b2, tw, tb)


def _edge_mlp(ea, w1, b1, w2, b2):
    E = ea.shape[0]
    grid = E // _EBLK

    def body(eb, w1b, b1b, w2b, b2b, out):
        hid = _silu(_dot(eb[...], w1b[...]) + b1b[...])
        out[...] = _dot(hid, w2b[...]) + b2b[...]

    full = lambda shape: pl.BlockSpec(shape, lambda i: (0, 0))
    return pl.pallas_call(
        body,
        grid=(grid,),
        in_specs=[
            pl.BlockSpec((_EBLK, 2), lambda i: (i, 0)),
            full((2, H)), full((1, H)), full((H, H)), full((1, H)),
        ],
        out_specs=pl.BlockSpec((_EBLK, H), lambda i: (i, 0)),
        out_shape=jax.ShapeDtypeStruct((E, H), jnp.float32),
    )(ea, w1, b1, w2, b2)


def _layer_mlp(h, aggp, w1, b1, w2, b2):
    N = h.shape[0]
    grid = N // _NBLK

    def body(hb, ab, w1b, b1b, w2b, b2b, out):
        z = hb[...] + ab[...]
        hid = _silu(_dot(z, w1b[...]) + b1b[...])
        out[...] = _silu(_dot(hid, w2b[...]) + b2b[...])

    full = lambda shape: pl.BlockSpec(shape, lambda i: (0, 0))
    return pl.pallas_call(
        body,
        grid=(grid,),
        in_specs=[
            pl.BlockSpec((_NBLK, H), lambda i: (i, 0)),
            pl.BlockSpec((_NBLK, H), lambda i: (i, 0)),
            full((H, H)), full((1, H)), full((H, H)), full((1, H)),
        ],
        out_specs=pl.BlockSpec((_NBLK, H), lambda i: (i, 0)),
        out_shape=jax.ShapeDtypeStruct((N, H), jnp.float32),
    )(h, aggp, w1, b1, w2, b2)


def _final_mlp(h, aggp, w1, b1, w2, b2, fw1, fb1, fw2, fb2):
    N = h.shape[0]
    grid = N // _NBLK

    def body(hb, ab, w1b, b1b, w2b, b2b, fw1b, fb1b, fw2b, fb2b, out):
        z = hb[...] + ab[...]
        hid = _silu(_dot(z, w1b[...]) + b1b[...])
        h3 = _silu(_dot(hid, w2b[...]) + b2b[...])
        fh = _silu(_dot(h3, fw1b[...]) + fb1b[...])
        out[...] = _dot(fh, fw2b[...]) + fb2b[...]

    full = lambda shape: pl.BlockSpec(shape, lambda i: (0, 0))
    return pl.pallas_call(
        body,
        grid=(grid,),
        in_specs=[
            pl.BlockSpec((_NBLK, H), lambda i: (i, 0)),
            pl.BlockSpec((_NBLK, H), lambda i: (i, 0)),
            full((H, H)), full((1, H)), full((H, H)), full((1, H)),
            full((H, H)), full((1, H)), full((H, 1)), full((1, 1)),
        ],
        out_specs=pl.BlockSpec((_NBLK, 1), lambda i: (i, 0)),
        out_shape=jax.ShapeDtypeStruct((N, 1), jnp.float32),
    )(h, aggp, w1, b1, w2, b2, fw1, fb1, fw2, fb2)


def kernel(x_t, t, condition, edge_index, edge_attr, batch, params):
    N = x_t.shape[0]
    E = edge_index.shape[1]
    p = params

    half = 16
    freq = jnp.exp(jnp.arange(half, dtype=jnp.float32)
                   * (-math.log(10000.0) / (half - 1)))
    ang = t[:, None].astype(jnp.float32) * freq[None, :]
    te = jnp.concatenate([jnp.sin(ang), jnp.cos(ang)], axis=-1)

    r1 = lambda v: v.reshape(1, -1)
    batch2 = batch.reshape(N, 1)

    x_in = jnp.concatenate([x_t, condition], axis=-1)
    h = _node_encoder(
        x_in, batch2, te,
        p['node_w1'], r1(p['node_b1']),
        p['node_w2'], r1(p['node_b2']),
        p['time_w'], r1(p['time_b']),
    )
    ef = _edge_mlp(edge_attr, p['edge_w1'], r1(p['edge_b1']),
                   p['edge_w2'], r1(p['edge_b2']))

    src, dst = edge_index[0], edge_index[1]
    for name in ('c1', 'c2'):
        aggp = _sc_scatter(h, ef, src, dst, N=N, E=E)
        h = _layer_mlp(h, aggp, p[name + '_w1'], r1(p[name + '_b1']),
                       p[name + '_w2'], r1(p[name + '_b2']))

    aggp = _sc_scatter(h, ef, src, dst, N=N, E=E)
    return _final_mlp(h, aggp, p['c3_w1'], r1(p['c3_b1']),
                      p['c3_w2'], r1(p['c3_b2']),
                      p['f_w1'], r1(p['f_b1']),
                      p['f_w2'], jnp.full((1, 1), p['f_b2'][0]))
